# Initial kernel scaffold; baseline (speedup 1.0000x reference)
import jax
import jax.numpy as jnp
from jax.experimental import pallas as pl


def kernel(x, edge_index, batch, edge_attr, node_W, node_b, edge_W, edge_b, eeW, eeb, preW, preb, postW, postb, linW, linb, bn_g, bn_b, headW1, headb1, headW2, headb2):
    raise NotImplementedError("write your pallas kernel here")


# speedup vs baseline: 1.4619x; 1.4619x over previous
import functools

import numpy as np
import jax
import jax.numpy as jnp
from jax import lax
from jax.experimental import pallas as pl
from jax.experimental.pallas import tpu as pltpu
from jax.experimental.pallas import tpu_sc as plsc

N = 10000
E = 320000
IE = 16
H = 128
L = 4
NG = 64

BLK = 1000
QBLK = 4000
NW = 32
C = 128
BIG = 3.0e38

_HIST = np.array([1.0, 2.0, 3.0, 4.0, 5.0, 6.0])
AVG_LOG = float((np.log(np.arange(6.0) + 1.0) * _HIST).sum() / _HIST.sum())


def _extract(v, lanemask, nbits):
    out = jnp.int32(0)
    for b in range(nbits):
        bit = jnp.any(jnp.logical_and(lanemask, ((v >> b) & 1) == 1))
        out = out + jnp.where(bit, jnp.int32(1 << b), jnp.int32(0))
    return out


def _sc_seg_body(hs_hbm, q_hbm, src_hbm, dst_hbm, bounds_hbm, out_hbm,
                 bounds_v, idx_v, dst_v, rows_v, q_v, acc, sem):
    cid = lax.axis_index("c")
    sid = lax.axis_index("s")
    wid = sid * 2 + cid
    lane = lax.iota(jnp.int32, 16)

    pltpu.sync_copy(bounds_hbm, bounds_v)
    wb = (wid // 16) * 16
    e0 = _extract(bounds_v[pl.ds(wb, 16)], lane == wid - wb, 19)
    wb1 = ((wid + 1) // 16) * 16
    e1 = _extract(bounds_v[pl.ds(wb1, 16)], lane == wid + 1 - wb1, 19)

    k0 = e0 // C
    k1 = (e1 + (C - 1)) // C

    def init_acc():
        z = jnp.zeros((16,), jnp.float32)
        p = jnp.full((16,), BIG, jnp.float32)
        m = jnp.full((16,), -BIG, jnp.float32)
        for s in range(8):
            acc[pl.ds(s * 16, 16)] = z
            acc[pl.ds(H + s * 16, 16)] = z
            acc[pl.ds(2 * H + s * 16, 16)] = p
            acc[pl.ds(3 * H + s * 16, 16)] = m

    def flush(cur):
        pltpu.sync_copy(acc, out_hbm.at[pl.ds(cur * (4 * H), 4 * H)])

    def chunk(k, cur):
        off = k * C
        pltpu.sync_copy(src_hbm.at[pl.ds(off, C)], idx_v)
        pltpu.sync_copy(dst_hbm.at[pl.ds(off, C)], dst_v)
        pltpu.sync_copy(q_hbm.at[pl.ds(off, C)], q_v)
        pltpu.async_copy(hs_hbm.at[idx_v], rows_v, sem).wait()
        j0 = jnp.maximum(e0 - off, 0)
        j1 = jnp.minimum(e1 - off, C)

        def edge(j, cur):
            jb = (j // 16) * 16
            dv = dst_v[pl.ds(jb, 16)]
            lm = lane == j - jb
            changed = jnp.any(jnp.logical_and(lm, dv != cur))

            @pl.when(jnp.logical_and(changed, cur >= 0))
            def _():
                flush(cur)

            @pl.when(changed)
            def _():
                init_acc()

            d = jnp.where(changed, _extract(dv, lm, 14), cur)

            for s in range(8):
                sl = pl.ds(s * 16, 16)
                u = rows_v[j, sl] + q_v[j, sl]
                plsc.addupdate(acc.at[pl.ds(s * 16, 16)], u)
                plsc.addupdate(acc.at[pl.ds(H + s * 16, 16)], u * u)
                sl_mn = pl.ds(2 * H + s * 16, 16)
                acc[sl_mn] = jnp.minimum(acc[sl_mn], u)
                sl_mx = pl.ds(3 * H + s * 16, 16)
                acc[sl_mx] = jnp.maximum(acc[sl_mx], u)
            return d

        return lax.fori_loop(j0, j1, edge, cur)

    cur = lax.fori_loop(k0, k1, chunk, jnp.int32(-1))

    @pl.when(cur >= 0)
    def _():
        flush(cur)


def _seg_stats_sc(hs, q, src_s, dst_s, bounds):
    mesh = plsc.VectorSubcoreMesh(core_axis_name="c", subcore_axis_name="s")
    f = functools.partial(
        pl.kernel,
        out_type=jax.ShapeDtypeStruct((N * 4 * H,), jnp.float32),
        mesh=mesh,
        compiler_params=pltpu.CompilerParams(needs_layout_passes=False),
        scratch_types=[
            pltpu.VMEM((64,), jnp.int32),
            pltpu.VMEM((C,), jnp.int32),
            pltpu.VMEM((C,), jnp.int32),
            pltpu.VMEM((C, H), jnp.float32),
            pltpu.VMEM((C, H), jnp.float32),
            pltpu.VMEM((4 * H,), jnp.float32),
            pltpu.SemaphoreType.DMA,
        ],
    )(_sc_seg_body)
    return f(hs, q, src_s, dst_s, bounds).reshape(N, 4 * H)


def _q_body(ea_ref, Ql_ref, cl_ref, out_ref):
    out_ref[...] = jnp.dot(ea_ref[...], Ql_ref[...],
                           preferred_element_type=jnp.float32) + cl_ref[...]


def _edge_q(ea_s, Ql, cl):
    return pl.pallas_call(
        _q_body,
        grid=(E // QBLK,),
        in_specs=[
            pl.BlockSpec((QBLK, IE), lambda i: (i, 0)),
            pl.BlockSpec((IE, H), lambda i: (0, 0)),
            pl.BlockSpec((1, H), lambda i: (0, 0)),
        ],
        out_specs=pl.BlockSpec((QBLK, H), lambda i: (i, 0)),
        out_shape=jax.ShapeDtypeStruct((E, H), jnp.float32),
    )(ea_s, Ql, cl)


def _layer_body(U_ref, h_ref, cnt_ref, preWa_ref, postW_ref,
                linW_ref, linb2_ref, out_ref, csum_ref, csq_ref):
    i = pl.program_id(0)
    cnt = cnt_ref[...]
    deg = jnp.maximum(cnt, 1.0)
    logd = jnp.log(deg + 1.0)
    sc1 = logd / AVG_LOG
    sc2 = AVG_LOG / logd
    mask = cnt > 0.0
    h = h_ref[...]
    hd = jnp.dot(h, preWa_ref[...], preferred_element_type=jnp.float32)
    su = U_ref[:, 0:H]
    sq = U_ref[:, H:2 * H]
    mnu = U_ref[:, 2 * H:3 * H]
    mxu = U_ref[:, 3 * H:4 * H]
    mean_u = su / deg
    msq_u = sq / deg
    mean = jnp.where(mask, hd + mean_u, 0.0)
    var_u = jax.nn.relu(msq_u - mean_u * mean_u)
    std = jnp.sqrt(jnp.where(mask, var_u, 0.0) + 1e-5)
    mn = jnp.where(mask, hd + mnu, 0.0)
    mx = jnp.where(mask, hd + mxu, 0.0)
    agg = jnp.concatenate([mean, mn, mx, std], axis=1)
    t = jnp.dot(h, postW_ref[0:H, :], preferred_element_type=jnp.float32)
    t += jnp.dot(agg, postW_ref[H:5 * H, :], preferred_element_type=jnp.float32)
    t += sc1 * jnp.dot(agg, postW_ref[5 * H:9 * H, :],
                       preferred_element_type=jnp.float32)
    t += sc2 * jnp.dot(agg, postW_ref[9 * H:13 * H, :],
                       preferred_element_type=jnp.float32)
    out = jnp.dot(t, linW_ref[...], preferred_element_type=jnp.float32)
    out = out + linb2_ref[...]
    out_ref[...] = out

    @pl.when(i == 0)
    def _():
        csum_ref[...] = jnp.zeros_like(csum_ref)
        csq_ref[...] = jnp.zeros_like(csq_ref)

    csum_ref[...] += jnp.sum(out, axis=0, keepdims=True)
    csq_ref[...] += jnp.sum(out * out, axis=0, keepdims=True)


def _dense_layer(U, h, cnt, preWa, postW, linW, linb2):
    return pl.pallas_call(
        _layer_body,
        grid=(N // BLK,),
        in_specs=[
            pl.BlockSpec((BLK, 4 * H), lambda i: (i, 0)),
            pl.BlockSpec((BLK, H), lambda i: (i, 0)),
            pl.BlockSpec((BLK, 1), lambda i: (i, 0)),
            pl.BlockSpec((H, H), lambda i: (0, 0)),
            pl.BlockSpec((13 * H, H), lambda i: (0, 0)),
            pl.BlockSpec((H, H), lambda i: (0, 0)),
            pl.BlockSpec((1, H), lambda i: (0, 0)),
        ],
        out_specs=[
            pl.BlockSpec((BLK, H), lambda i: (i, 0)),
            pl.BlockSpec((1, H), lambda i: (0, 0)),
            pl.BlockSpec((1, H), lambda i: (0, 0)),
        ],
        out_shape=[
            jax.ShapeDtypeStruct((N, H), jnp.float32),
            jax.ShapeDtypeStruct((1, H), jnp.float32),
            jax.ShapeDtypeStruct((1, H), jnp.float32),
        ],
        compiler_params=pltpu.CompilerParams(
            dimension_semantics=("arbitrary",)),
    )(U, h, cnt, preWa, postW, linW, linb2)


def kernel(x, edge_index, batch, edge_attr, node_W, node_b, edge_W, edge_b,
           eeW, eeb, preW, preb, postW, postb, linW, linb, bn_g, bn_b,
           headW1, headb1, headW2, headb2):
    src = edge_index[0]
    dst = edge_index[1]

    dst_s, src_s, perm = lax.sort(
        (dst, src, jnp.arange(E, dtype=jnp.int32)), num_keys=1)
    ea_s = edge_attr[perm]

    row_starts = jnp.searchsorted(dst_s, jnp.arange(N + 1, dtype=jnp.int32)
                                  ).astype(jnp.int32)
    counts = (row_starts[1:] - row_starts[:-1]).astype(jnp.float32)
    cnt = counts.reshape(N, 1)
    targets = (jnp.arange(NW, dtype=jnp.int32) * (E // NW))
    starts = row_starts[dst_s[targets]]
    bounds = jnp.concatenate(
        [starts, jnp.full((32,), E, jnp.int32)]).astype(jnp.int32)

    h = x @ node_W + node_b

    for l in range(L):
        preWa = preW[l][0:H]
        preWb = preW[l][H:2 * H]
        preWc = preW[l][2 * H:3 * H]
        M = eeW[l] @ preWc
        Ql = edge_W @ M
        cl = (edge_b @ M + eeb[l] @ preWc + preb[l]).reshape(1, H)
        q = _edge_q(ea_s, Ql, cl)
        hs = h @ preWb
        U = _seg_stats_sc(hs, q, src_s, dst_s, bounds)

        linb2 = (postb[l] @ linW[l] + linb[l]).reshape(1, H)
        out, cs, cq = _dense_layer(U, h, cnt, preWa, postW[l], linW[l], linb2)
        mu = cs[0] / N
        var = cq[0] / N - mu * mu
        out = bn_g[l] * (out - mu) * lax.rsqrt(var + 1e-5) + bn_b[l]
        h = jax.nn.relu(out)

    g = jax.ops.segment_sum(h, batch, num_segments=NG)
    g = jax.nn.relu(g @ headW1 + headb1)
    return g @ headW2 + headb2


# speedup vs baseline: 1.5398x; 1.0533x over previous
import functools

import numpy as np
import jax
import jax.numpy as jnp
from jax import lax
from jax.experimental import pallas as pl
from jax.experimental.pallas import tpu as pltpu
from jax.experimental.pallas import tpu_sc as plsc

N = 10000
E = 320000
IE = 16
H = 128
L = 4
NG = 64

BLK = 1000
QBLK = 4000
NW = 32
C = 128
BIG = 3.0e38

_HIST = np.array([1.0, 2.0, 3.0, 4.0, 5.0, 6.0])
AVG_LOG = float((np.log(np.arange(6.0) + 1.0) * _HIST).sum() / _HIST.sum())


def _extract(v, lanemask, nbits):
    out = jnp.int32(0)
    for b in range(nbits):
        bit = jnp.any(jnp.logical_and(lanemask, ((v >> b) & 1) == 1))
        out = out + jnp.where(bit, jnp.int32(1 << b), jnp.int32(0))
    return out


def _sc_seg_body(hs_hbm, q_hbm, src_hbm, dst_hbm, bounds_hbm, out_hbm,
                 bounds_v, idx_v, dst_v, rows_v, q_v, acc, sem):
    cid = lax.axis_index("c")
    sid = lax.axis_index("s")
    wid = sid * 2 + cid
    lane = lax.iota(jnp.int32, 16)

    pltpu.sync_copy(bounds_hbm, bounds_v)
    wb = (wid // 16) * 16
    e0 = _extract(bounds_v[pl.ds(wb, 16)], lane == wid - wb, 19)
    wb1 = ((wid + 1) // 16) * 16
    e1 = _extract(bounds_v[pl.ds(wb1, 16)], lane == wid + 1 - wb1, 19)

    k0 = e0 // C
    k1 = (e1 + (C - 1)) // C

    def init_acc():
        z = jnp.zeros((16,), jnp.float32)
        p = jnp.full((16,), BIG, jnp.float32)
        m = jnp.full((16,), -BIG, jnp.float32)
        for s in range(8):
            acc[pl.ds(s * 16, 16)] = z
            acc[pl.ds(H + s * 16, 16)] = z
            acc[pl.ds(2 * H + s * 16, 16)] = p
            acc[pl.ds(3 * H + s * 16, 16)] = m

    def flush(cur):
        pltpu.sync_copy(acc, out_hbm.at[pl.ds(cur * (4 * H), 4 * H)])

    def chunk(k, cur):
        off = k * C
        pltpu.sync_copy(src_hbm.at[pl.ds(off, C)], idx_v)
        pltpu.sync_copy(dst_hbm.at[pl.ds(off, C)], dst_v)
        pltpu.sync_copy(q_hbm.at[pl.ds(off, C)], q_v)
        pltpu.async_copy(hs_hbm.at[idx_v], rows_v, sem).wait()
        j0 = jnp.maximum(e0 - off, 0)
        j1 = jnp.minimum(e1 - off, C)

        def edge(j, cur):
            jb = (j // 16) * 16
            dv = dst_v[pl.ds(jb, 16)]
            lm = lane == j - jb
            changed = jnp.any(jnp.logical_and(lm, dv != cur))

            @pl.when(jnp.logical_and(changed, cur >= 0))
            def _():
                flush(cur)

            @pl.when(changed)
            def _():
                init_acc()

            d = lax.cond(changed, lambda: _extract(dv, lm, 14), lambda: cur)

            for s in range(8):
                sl = pl.ds(s * 16, 16)
                u = rows_v[j, sl] + q_v[j, sl]
                plsc.addupdate(acc.at[pl.ds(s * 16, 16)], u)
                plsc.addupdate(acc.at[pl.ds(H + s * 16, 16)], u * u)
                sl_mn = pl.ds(2 * H + s * 16, 16)
                acc[sl_mn] = jnp.minimum(acc[sl_mn], u)
                sl_mx = pl.ds(3 * H + s * 16, 16)
                acc[sl_mx] = jnp.maximum(acc[sl_mx], u)
            return d

        return lax.fori_loop(j0, j1, edge, cur)

    cur = lax.fori_loop(k0, k1, chunk, jnp.int32(-1))

    @pl.when(cur >= 0)
    def _():
        flush(cur)


def _seg_stats_sc(hs, q, src_s, dst_s, bounds):
    mesh = plsc.VectorSubcoreMesh(core_axis_name="c", subcore_axis_name="s")
    f = functools.partial(
        pl.kernel,
        out_type=jax.ShapeDtypeStruct((N * 4 * H,), jnp.float32),
        mesh=mesh,
        compiler_params=pltpu.CompilerParams(needs_layout_passes=False),
        scratch_types=[
            pltpu.VMEM((64,), jnp.int32),
            pltpu.VMEM((C,), jnp.int32),
            pltpu.VMEM((C,), jnp.int32),
            pltpu.VMEM((C, H), jnp.float32),
            pltpu.VMEM((C, H), jnp.float32),
            pltpu.VMEM((4 * H,), jnp.float32),
            pltpu.SemaphoreType.DMA,
        ],
    )(_sc_seg_body)
    return f(hs, q, src_s, dst_s, bounds).reshape(N, 4 * H)


def _q_body(ea_ref, Ql_ref, cl_ref, out_ref):
    out_ref[...] = jnp.dot(ea_ref[...], Ql_ref[...],
                           preferred_element_type=jnp.float32) + cl_ref[...]


def _edge_q(ea_s, Ql, cl):
    return pl.pallas_call(
        _q_body,
        grid=(E // QBLK,),
        in_specs=[
            pl.BlockSpec((QBLK, IE), lambda i: (i, 0)),
            pl.BlockSpec((IE, H), lambda i: (0, 0)),
            pl.BlockSpec((1, H), lambda i: (0, 0)),
        ],
        out_specs=pl.BlockSpec((QBLK, H), lambda i: (i, 0)),
        out_shape=jax.ShapeDtypeStruct((E, H), jnp.float32),
    )(ea_s, Ql, cl)


def _layer_body(U_ref, h_ref, cnt_ref, preWa_ref, postW_ref,
                linW_ref, linb2_ref, out_ref, csum_ref, csq_ref):
    i = pl.program_id(0)
    cnt = cnt_ref[...]
    deg = jnp.maximum(cnt, 1.0)
    logd = jnp.log(deg + 1.0)
    sc1 = logd / AVG_LOG
    sc2 = AVG_LOG / logd
    mask = cnt > 0.0
    h = h_ref[...]
    hd = jnp.dot(h, preWa_ref[...], preferred_element_type=jnp.float32)
    su = U_ref[:, 0:H]
    sq = U_ref[:, H:2 * H]
    mnu = U_ref[:, 2 * H:3 * H]
    mxu = U_ref[:, 3 * H:4 * H]
    mean_u = su / deg
    msq_u = sq / deg
    mean = jnp.where(mask, hd + mean_u, 0.0)
    var_u = jax.nn.relu(msq_u - mean_u * mean_u)
    std = jnp.sqrt(jnp.where(mask, var_u, 0.0) + 1e-5)
    mn = jnp.where(mask, hd + mnu, 0.0)
    mx = jnp.where(mask, hd + mxu, 0.0)
    agg = jnp.concatenate([mean, mn, mx, std], axis=1)
    t = jnp.dot(h, postW_ref[0:H, :], preferred_element_type=jnp.float32)
    t += jnp.dot(agg, postW_ref[H:5 * H, :], preferred_element_type=jnp.float32)
    t += sc1 * jnp.dot(agg, postW_ref[5 * H:9 * H, :],
                       preferred_element_type=jnp.float32)
    t += sc2 * jnp.dot(agg, postW_ref[9 * H:13 * H, :],
                       preferred_element_type=jnp.float32)
    out = jnp.dot(t, linW_ref[...], preferred_element_type=jnp.float32)
    out = out + linb2_ref[...]
    out_ref[...] = out

    @pl.when(i == 0)
    def _():
        csum_ref[...] = jnp.zeros_like(csum_ref)
        csq_ref[...] = jnp.zeros_like(csq_ref)

    csum_ref[...] += jnp.sum(out, axis=0, keepdims=True)
    csq_ref[...] += jnp.sum(out * out, axis=0, keepdims=True)


def _dense_layer(U, h, cnt, preWa, postW, linW, linb2):
    return pl.pallas_call(
        _layer_body,
        grid=(N // BLK,),
        in_specs=[
            pl.BlockSpec((BLK, 4 * H), lambda i: (i, 0)),
            pl.BlockSpec((BLK, H), lambda i: (i, 0)),
            pl.BlockSpec((BLK, 1), lambda i: (i, 0)),
            pl.BlockSpec((H, H), lambda i: (0, 0)),
            pl.BlockSpec((13 * H, H), lambda i: (0, 0)),
            pl.BlockSpec((H, H), lambda i: (0, 0)),
            pl.BlockSpec((1, H), lambda i: (0, 0)),
        ],
        out_specs=[
            pl.BlockSpec((BLK, H), lambda i: (i, 0)),
            pl.BlockSpec((1, H), lambda i: (0, 0)),
            pl.BlockSpec((1, H), lambda i: (0, 0)),
        ],
        out_shape=[
            jax.ShapeDtypeStruct((N, H), jnp.float32),
            jax.ShapeDtypeStruct((1, H), jnp.float32),
            jax.ShapeDtypeStruct((1, H), jnp.float32),
        ],
        compiler_params=pltpu.CompilerParams(
            dimension_semantics=("arbitrary",)),
    )(U, h, cnt, preWa, postW, linW, linb2)


def kernel(x, edge_index, batch, edge_attr, node_W, node_b, edge_W, edge_b,
           eeW, eeb, preW, preb, postW, postb, linW, linb, bn_g, bn_b,
           headW1, headb1, headW2, headb2):
    src = edge_index[0]
    dst = edge_index[1]

    dst_s, src_s, perm = lax.sort(
        (dst, src, jnp.arange(E, dtype=jnp.int32)), num_keys=1)
    ea_s = edge_attr[perm]

    row_starts = jnp.searchsorted(dst_s, jnp.arange(N + 1, dtype=jnp.int32)
                                  ).astype(jnp.int32)
    counts = (row_starts[1:] - row_starts[:-1]).astype(jnp.float32)
    cnt = counts.reshape(N, 1)
    targets = (jnp.arange(NW, dtype=jnp.int32) * (E // NW))
    starts = row_starts[dst_s[targets]]
    bounds = jnp.concatenate(
        [starts, jnp.full((32,), E, jnp.int32)]).astype(jnp.int32)

    h = x @ node_W + node_b

    for l in range(L):
        preWa = preW[l][0:H]
        preWb = preW[l][H:2 * H]
        preWc = preW[l][2 * H:3 * H]
        M = eeW[l] @ preWc
        Ql = edge_W @ M
        cl = (edge_b @ M + eeb[l] @ preWc + preb[l]).reshape(1, H)
        q = _edge_q(ea_s, Ql, cl)
        hs = h @ preWb
        U = _seg_stats_sc(hs, q, src_s, dst_s, bounds)

        linb2 = (postb[l] @ linW[l] + linb[l]).reshape(1, H)
        out, cs, cq = _dense_layer(U, h, cnt, preWa, postW[l], linW[l], linb2)
        mu = cs[0] / N
        var = cq[0] / N - mu * mu
        out = bn_g[l] * (out - mu) * lax.rsqrt(var + 1e-5) + bn_b[l]
        h = jax.nn.relu(out)

    g = jax.ops.segment_sum(h, batch, num_segments=NG)
    g = jax.nn.relu(g @ headW1 + headb1)
    return g @ headW2 + headb2


# speedup vs baseline: 1.5904x; 1.0329x over previous
import functools

import numpy as np
import jax
import jax.numpy as jnp
from jax import lax
from jax.experimental import pallas as pl
from jax.experimental.pallas import tpu as pltpu
from jax.experimental.pallas import tpu_sc as plsc

N = 10000
E = 320000
IE = 16
H = 128
L = 4
NG = 64

BLK = 1000
QBLK = 4000
NW = 32
C = 128
BIG = 3.0e38

_HIST = np.array([1.0, 2.0, 3.0, 4.0, 5.0, 6.0])
AVG_LOG = float((np.log(np.arange(6.0) + 1.0) * _HIST).sum() / _HIST.sum())


def _extract(v, lanemask, nbits):
    out = jnp.int32(0)
    for b in range(nbits):
        bit = jnp.any(jnp.logical_and(lanemask, ((v >> b) & 1) == 1))
        out = out + jnp.where(bit, jnp.int32(1 << b), jnp.int32(0))
    return out


def _sc_seg_body(hs_hbm, q_hbm, src_hbm, dst_hbm, bounds_hbm, out_hbm,
                 bounds_v, idx_v, dst_v, rows_v, q_v, acc, sem, sem2):
    cid = lax.axis_index("c")
    sid = lax.axis_index("s")
    wid = sid * 2 + cid
    lane = lax.iota(jnp.int32, 16)

    pltpu.sync_copy(bounds_hbm, bounds_v)
    wb = (wid // 16) * 16
    e0 = _extract(bounds_v[pl.ds(wb, 16)], lane == wid - wb, 19)
    wb1 = ((wid + 1) // 16) * 16
    e1 = _extract(bounds_v[pl.ds(wb1, 16)], lane == wid + 1 - wb1, 19)

    k0 = e0 // C
    k1 = (e1 + (C - 1)) // C

    def init_acc():
        z = jnp.zeros((16,), jnp.float32)
        p = jnp.full((16,), BIG, jnp.float32)
        m = jnp.full((16,), -BIG, jnp.float32)
        for s in range(8):
            acc[pl.ds(s * 16, 16)] = z
            acc[pl.ds(H + s * 16, 16)] = z
            acc[pl.ds(2 * H + s * 16, 16)] = p
            acc[pl.ds(3 * H + s * 16, 16)] = m

    def flush(cur):
        pltpu.sync_copy(acc, out_hbm.at[pl.ds(cur * (4 * H), 4 * H)])

    def chunk(k, cur):
        off = k * C
        pltpu.sync_copy(src_hbm.at[pl.ds(off, C)], idx_v)
        cg = pltpu.async_copy(hs_hbm.at[idx_v], rows_v, sem)
        cq = pltpu.async_copy(q_hbm.at[pl.ds(off, C)], q_v, sem2)
        pltpu.sync_copy(dst_hbm.at[pl.ds(off, C)], dst_v)
        cg.wait()
        cq.wait()
        j0 = jnp.maximum(e0 - off, 0)
        j1 = jnp.minimum(e1 - off, C)

        def edge(j, cur):
            jb = (j // 16) * 16
            dv = dst_v[pl.ds(jb, 16)]
            lm = lane == j - jb
            changed = jnp.any(jnp.logical_and(lm, dv != cur))

            @pl.when(jnp.logical_and(changed, cur >= 0))
            def _():
                flush(cur)

            @pl.when(changed)
            def _():
                init_acc()

            d = lax.cond(changed, lambda: _extract(dv, lm, 14), lambda: cur)

            for s in range(8):
                sl = pl.ds(s * 16, 16)
                u = rows_v[j, sl] + q_v[j, sl]
                plsc.addupdate(acc.at[pl.ds(s * 16, 16)], u)
                plsc.addupdate(acc.at[pl.ds(H + s * 16, 16)], u * u)
                sl_mn = pl.ds(2 * H + s * 16, 16)
                acc[sl_mn] = jnp.minimum(acc[sl_mn], u)
                sl_mx = pl.ds(3 * H + s * 16, 16)
                acc[sl_mx] = jnp.maximum(acc[sl_mx], u)
            return d

        return lax.fori_loop(j0, j1, edge, cur)

    cur = lax.fori_loop(k0, k1, chunk, jnp.int32(-1))

    @pl.when(cur >= 0)
    def _():
        flush(cur)


def _seg_stats_sc(hs, q, src_s, dst_s, bounds):
    mesh = plsc.VectorSubcoreMesh(core_axis_name="c", subcore_axis_name="s")
    f = functools.partial(
        pl.kernel,
        out_type=jax.ShapeDtypeStruct((N * 4 * H,), jnp.float32),
        mesh=mesh,
        compiler_params=pltpu.CompilerParams(needs_layout_passes=False),
        scratch_types=[
            pltpu.VMEM((64,), jnp.int32),
            pltpu.VMEM((C,), jnp.int32),
            pltpu.VMEM((C,), jnp.int32),
            pltpu.VMEM((C, H), jnp.float32),
            pltpu.VMEM((C, H), jnp.float32),
            pltpu.VMEM((4 * H,), jnp.float32),
            pltpu.SemaphoreType.DMA,
            pltpu.SemaphoreType.DMA,
        ],
    )(_sc_seg_body)
    return f(hs, q, src_s, dst_s, bounds).reshape(N, 4 * H)


def _q_body(ea_ref, Ql_ref, cl_ref, out_ref):
    out_ref[...] = jnp.dot(ea_ref[...], Ql_ref[...],
                           preferred_element_type=jnp.float32) + cl_ref[...]


def _edge_q(ea_s, Ql, cl):
    return pl.pallas_call(
        _q_body,
        grid=(E // QBLK,),
        in_specs=[
            pl.BlockSpec((QBLK, IE), lambda i: (i, 0)),
            pl.BlockSpec((IE, H), lambda i: (0, 0)),
            pl.BlockSpec((1, H), lambda i: (0, 0)),
        ],
        out_specs=pl.BlockSpec((QBLK, H), lambda i: (i, 0)),
        out_shape=jax.ShapeDtypeStruct((E, H), jnp.float32),
    )(ea_s, Ql, cl)


def _layer_body(U_ref, h_ref, cnt_ref, preWa_ref, postW_ref,
                linW_ref, linb2_ref, out_ref, csum_ref, csq_ref):
    i = pl.program_id(0)
    cnt = cnt_ref[...]
    deg = jnp.maximum(cnt, 1.0)
    logd = jnp.log(deg + 1.0)
    sc1 = logd / AVG_LOG
    sc2 = AVG_LOG / logd
    mask = cnt > 0.0
    h = h_ref[...]
    hd = jnp.dot(h, preWa_ref[...], preferred_element_type=jnp.float32)
    su = U_ref[:, 0:H]
    sq = U_ref[:, H:2 * H]
    mnu = U_ref[:, 2 * H:3 * H]
    mxu = U_ref[:, 3 * H:4 * H]
    mean_u = su / deg
    msq_u = sq / deg
    mean = jnp.where(mask, hd + mean_u, 0.0)
    var_u = jax.nn.relu(msq_u - mean_u * mean_u)
    std = jnp.sqrt(jnp.where(mask, var_u, 0.0) + 1e-5)
    mn = jnp.where(mask, hd + mnu, 0.0)
    mx = jnp.where(mask, hd + mxu, 0.0)
    agg = jnp.concatenate([mean, mn, mx, std], axis=1)
    t = jnp.dot(h, postW_ref[0:H, :], preferred_element_type=jnp.float32)
    t += jnp.dot(agg, postW_ref[H:5 * H, :], preferred_element_type=jnp.float32)
    t += sc1 * jnp.dot(agg, postW_ref[5 * H:9 * H, :],
                       preferred_element_type=jnp.float32)
    t += sc2 * jnp.dot(agg, postW_ref[9 * H:13 * H, :],
                       preferred_element_type=jnp.float32)
    out = jnp.dot(t, linW_ref[...], preferred_element_type=jnp.float32)
    out = out + linb2_ref[...]
    out_ref[...] = out

    @pl.when(i == 0)
    def _():
        csum_ref[...] = jnp.zeros_like(csum_ref)
        csq_ref[...] = jnp.zeros_like(csq_ref)

    csum_ref[...] += jnp.sum(out, axis=0, keepdims=True)
    csq_ref[...] += jnp.sum(out * out, axis=0, keepdims=True)


def _dense_layer(U, h, cnt, preWa, postW, linW, linb2):
    return pl.pallas_call(
        _layer_body,
        grid=(N // BLK,),
        in_specs=[
            pl.BlockSpec((BLK, 4 * H), lambda i: (i, 0)),
            pl.BlockSpec((BLK, H), lambda i: (i, 0)),
            pl.BlockSpec((BLK, 1), lambda i: (i, 0)),
            pl.BlockSpec((H, H), lambda i: (0, 0)),
            pl.BlockSpec((13 * H, H), lambda i: (0, 0)),
            pl.BlockSpec((H, H), lambda i: (0, 0)),
            pl.BlockSpec((1, H), lambda i: (0, 0)),
        ],
        out_specs=[
            pl.BlockSpec((BLK, H), lambda i: (i, 0)),
            pl.BlockSpec((1, H), lambda i: (0, 0)),
            pl.BlockSpec((1, H), lambda i: (0, 0)),
        ],
        out_shape=[
            jax.ShapeDtypeStruct((N, H), jnp.float32),
            jax.ShapeDtypeStruct((1, H), jnp.float32),
            jax.ShapeDtypeStruct((1, H), jnp.float32),
        ],
        compiler_params=pltpu.CompilerParams(
            dimension_semantics=("arbitrary",)),
    )(U, h, cnt, preWa, postW, linW, linb2)


def kernel(x, edge_index, batch, edge_attr, node_W, node_b, edge_W, edge_b,
           eeW, eeb, preW, preb, postW, postb, linW, linb, bn_g, bn_b,
           headW1, headb1, headW2, headb2):
    src = edge_index[0]
    dst = edge_index[1]

    dst_s, src_s, perm = lax.sort(
        (dst, src, jnp.arange(E, dtype=jnp.int32)), num_keys=1)
    ea_s = edge_attr[perm]

    row_starts = jnp.searchsorted(dst_s, jnp.arange(N + 1, dtype=jnp.int32)
                                  ).astype(jnp.int32)
    counts = (row_starts[1:] - row_starts[:-1]).astype(jnp.float32)
    cnt = counts.reshape(N, 1)
    targets = (jnp.arange(NW, dtype=jnp.int32) * (E // NW))
    starts = row_starts[dst_s[targets]]
    bounds = jnp.concatenate(
        [starts, jnp.full((32,), E, jnp.int32)]).astype(jnp.int32)

    h = x @ node_W + node_b

    for l in range(L):
        preWa = preW[l][0:H]
        preWb = preW[l][H:2 * H]
        preWc = preW[l][2 * H:3 * H]
        M = eeW[l] @ preWc
        Ql = edge_W @ M
        cl = (edge_b @ M + eeb[l] @ preWc + preb[l]).reshape(1, H)
        q = _edge_q(ea_s, Ql, cl)
        hs = h @ preWb
        U = _seg_stats_sc(hs, q, src_s, dst_s, bounds)

        linb2 = (postb[l] @ linW[l] + linb[l]).reshape(1, H)
        out, cs, cq = _dense_layer(U, h, cnt, preWa, postW[l], linW[l], linb2)
        mu = cs[0] / N
        var = cq[0] / N - mu * mu
        out = bn_g[l] * (out - mu) * lax.rsqrt(var + 1e-5) + bn_b[l]
        h = jax.nn.relu(out)

    g = jax.ops.segment_sum(h, batch, num_segments=NG)
    g = jax.nn.relu(g @ headW1 + headb1)
    return g @ headW2 + headb2
